# Initial kernel scaffold; baseline (speedup 1.0000x reference)
import jax
import jax.numpy as jnp
from jax.experimental import pallas as pl


def kernel(hidden_states, gate_w, alpha, W1, b1, W2, b2):
    raise NotImplementedError("write your pallas kernel here")


# speedup vs baseline: 1.7644x; 1.7644x over previous
import functools

import jax
import jax.numpy as jnp
from jax import lax
from jax.experimental import pallas as pl
from jax.experimental.pallas import tpu as pltpu
from jax.experimental.pallas import tpu_sc as plsc

TILE = 128


def _gate_body(x_ref, gw_ref, alpha_ref, eid_ref, w_ref, *, n_experts):
    x = x_ref[...]
    logits = jnp.dot(x, gw_ref[...], preferred_element_type=jnp.float32)
    t, lanes = logits.shape
    col = lax.broadcasted_iota(jnp.int32, (t, lanes), 1)
    in_cols = col < n_experts
    logits = jnp.where(in_cols, logits, -1e30)
    lmax = jnp.max(logits, axis=1, keepdims=True)
    ssum = jnp.sum(jnp.where(in_cols, jnp.exp(logits - lmax), 0.0), axis=1,
                   keepdims=True)
    eidx = jnp.min(jnp.where(logits == lmax, col, n_experts), axis=1)
    alpha_sel = jnp.sum(
        jnp.where(col == eidx[:, None], alpha_ref[...], 0.0), axis=1)
    eid_ref[...] = eidx
    w_ref[...] = alpha_sel / ssum[:, 0]


def _gate(x, gate_w, alpha):
    t, h = x.shape
    e = gate_w.shape[1]
    gwp = jnp.pad(gate_w, ((0, 0), (0, 128 - e)))
    alphap = jnp.pad(alpha, (0, 128 - e)).reshape(1, 128)
    return pl.pallas_call(
        functools.partial(_gate_body, n_experts=e),
        out_shape=(jax.ShapeDtypeStruct((t,), jnp.int32),
                   jax.ShapeDtypeStruct((t,), jnp.float32)),
    )(x, gwp, alphap)


def _sc_gather_rows(table, idx):
    t, h = table.shape
    info = plsc.get_sparse_core_info()
    nw = info.num_cores * info.num_subcores
    b_per_w = t // nw
    mesh = plsc.VectorSubcoreMesh(core_axis_name="c", subcore_axis_name="s")

    @functools.partial(
        pl.kernel, mesh=mesh,
        out_type=jax.ShapeDtypeStruct((t, h), table.dtype),
        scratch_types=[
            pltpu.VMEM((b_per_w,), jnp.int32),
            pltpu.VMEM((b_per_w, h), table.dtype),
            pltpu.SemaphoreType.DMA,
        ],
    )
    def k(table_hbm, idx_hbm, out_hbm, idx_v, rows_v, sem):
        wid = lax.axis_index("s") * info.num_cores + lax.axis_index("c")
        base = wid * b_per_w
        pltpu.sync_copy(idx_hbm.at[pl.ds(base, b_per_w)], idx_v)
        pltpu.async_copy(table_hbm.at[idx_v], rows_v, sem).wait()
        pltpu.sync_copy(rows_v, out_hbm.at[pl.ds(base, b_per_w)])

    return k(table, idx)


def _moe_body(meta_ref, x_ref, w1_ref, b1_ref, w2_ref, b2_ref, eid_ref,
              wrow_ref, out_ref):
    s = pl.program_id(0)
    e = meta_ref[1, s]
    first = meta_ref[2, s]
    valid = meta_ref[3, s]

    @pl.when(first == 1)
    def _():
        out_ref[...] = jnp.zeros_like(out_ref)

    @pl.when(valid == 1)
    def _():
        x = x_ref[...]
        h = jnp.dot(x, w1_ref[0], preferred_element_type=jnp.float32)
        h = jax.nn.gelu(h + b1_ref[0])
        y = jnp.dot(h, w2_ref[0], preferred_element_type=jnp.float32)
        y = y + b2_ref[0]
        coef = jnp.where(eid_ref[0, 0, :] == e, wrow_ref[0, 0, :], 0.0)
        out_ref[...] += y * coef[:, None]


def _grouped_mlp(meta, x_sorted, W1, b1, W2, b2, eid3, w3, n_steps):
    t, h = x_sorted.shape
    e, _, f = W1.shape
    grid_spec = pltpu.PrefetchScalarGridSpec(
        num_scalar_prefetch=1,
        grid=(n_steps,),
        in_specs=[
            pl.BlockSpec((TILE, h), lambda s, m: (m[0, s], 0)),
            pl.BlockSpec((1, h, f), lambda s, m: (m[1, s], 0, 0)),
            pl.BlockSpec((1, 1, f), lambda s, m: (m[1, s], 0, 0)),
            pl.BlockSpec((1, f, h), lambda s, m: (m[1, s], 0, 0)),
            pl.BlockSpec((1, 1, h), lambda s, m: (m[1, s], 0, 0)),
            pl.BlockSpec((1, 1, TILE), lambda s, m: (m[0, s], 0, 0)),
            pl.BlockSpec((1, 1, TILE), lambda s, m: (m[0, s], 0, 0)),
        ],
        out_specs=pl.BlockSpec((TILE, h), lambda s, m: (m[0, s], 0)),
    )
    return pl.pallas_call(
        _moe_body,
        grid_spec=grid_spec,
        out_shape=jax.ShapeDtypeStruct((t, h), jnp.float32),
    )(meta, x_sorted, W1, b1.reshape(e, 1, f), W2, b2.reshape(e, 1, h),
      eid3, w3)


def _routing(eid, wcomb, n_experts, n_tiles, n_steps):
    t = eid.shape[0]
    i32 = jnp.int32
    toks = jnp.arange(t, dtype=i32)
    oh = (eid[:, None] == jnp.arange(n_experts, dtype=i32)[None, :])
    oh = oh.astype(i32)
    counts = oh.sum(axis=0)
    offsets = jnp.concatenate(
        [jnp.zeros((1,), i32), jnp.cumsum(counts)[:-1].astype(i32)])
    csum = jnp.cumsum(oh, axis=0) - oh
    rank = ((offsets[None, :] + csum) * oh).sum(axis=1).astype(i32)
    sort_idx = jnp.zeros((t,), i32).at[rank].set(toks)
    eid_sorted = jnp.zeros((t,), i32).at[rank].set(eid)
    w_sorted = jnp.zeros((t,), jnp.float32).at[rank].set(wcomb)

    er = eid_sorted.reshape(n_tiles, -1)
    first_e = er[:, 0]
    last_e = er[:, -1]
    nsteps = last_e - first_e + 1
    step_start = jnp.concatenate(
        [jnp.zeros((1,), i32), jnp.cumsum(nsteps)[:-1].astype(i32)])
    s_real = step_start[-1] + nsteps[-1]
    s_ar = jnp.arange(n_steps, dtype=i32)
    t_s = jnp.searchsorted(step_start, s_ar, side="right").astype(i32) - 1
    e_s = first_e[t_s] + (s_ar - step_start[t_s])
    valid = s_ar < s_real
    e_s = jnp.where(valid, e_s, last_e[-1]).astype(i32)
    first_flag = (valid & (s_ar == step_start[t_s])).astype(i32)
    meta = jnp.stack([t_s, e_s, first_flag, valid.astype(i32)])
    return meta, rank, sort_idx, eid_sorted, w_sorted


def kernel(hidden_states, gate_w, alpha, W1, b1, W2, b2):
    t, h = hidden_states.shape
    e = gate_w.shape[1]
    nt = t // TILE
    n_steps = nt + e - 1

    eid, wcomb = _gate(hidden_states, gate_w, alpha)
    meta, rank, sort_idx, eid_sorted, w_sorted = _routing(
        eid, wcomb, e, nt, n_steps)
    x_sorted = _sc_gather_rows(hidden_states, sort_idx)
    y_sorted = _grouped_mlp(
        meta, x_sorted, W1, b1, W2, b2,
        eid_sorted.reshape(nt, 1, TILE), w_sorted.reshape(nt, 1, TILE),
        n_steps)
    return _sc_gather_rows(y_sorted, rank)


# speedup vs baseline: 1.9490x; 1.1047x over previous
import functools

import jax
import jax.numpy as jnp
from jax import lax
from jax.experimental import pallas as pl
from jax.experimental.pallas import tpu as pltpu
from jax.experimental.pallas import tpu_sc as plsc

TILE = 128


def _gate_body(x_ref, gw_ref, alpha_ref, eid_ref, w_ref, *, n_experts):
    x = x_ref[...]
    logits = jnp.dot(x, gw_ref[...], preferred_element_type=jnp.float32)
    t, lanes = logits.shape
    col = lax.broadcasted_iota(jnp.int32, (t, lanes), 1)
    in_cols = col < n_experts
    logits = jnp.where(in_cols, logits, -1e30)
    lmax = jnp.max(logits, axis=1, keepdims=True)
    ssum = jnp.sum(jnp.where(in_cols, jnp.exp(logits - lmax), 0.0), axis=1,
                   keepdims=True)
    eidx = jnp.min(jnp.where(logits == lmax, col, n_experts), axis=1)
    alpha_sel = jnp.sum(
        jnp.where(col == eidx[:, None], alpha_ref[...], 0.0), axis=1)
    eid_ref[...] = eidx
    w_ref[...] = alpha_sel / ssum[:, 0]


def _gate(x, gate_w, alpha):
    t, h = x.shape
    e = gate_w.shape[1]
    gwp = jnp.pad(gate_w, ((0, 0), (0, 128 - e)))
    alphap = jnp.pad(alpha, (0, 128 - e)).reshape(1, 128)
    return pl.pallas_call(
        functools.partial(_gate_body, n_experts=e),
        out_shape=(jax.ShapeDtypeStruct((t,), jnp.int32),
                   jax.ShapeDtypeStruct((t,), jnp.float32)),
    )(x, gwp, alphap)


def _sc_gather_rows(table, idx):
    t, h = table.shape
    info = plsc.get_sparse_core_info()
    nw = info.num_cores * info.num_subcores
    b_per_w = t // nw
    mesh = plsc.VectorSubcoreMesh(core_axis_name="c", subcore_axis_name="s")

    @functools.partial(
        pl.kernel, mesh=mesh,
        out_type=jax.ShapeDtypeStruct((t, h), table.dtype),
        scratch_types=[
            pltpu.VMEM((b_per_w,), jnp.int32),
            pltpu.VMEM((b_per_w, h), table.dtype),
            pltpu.SemaphoreType.DMA,
        ],
    )
    def k(table_hbm, idx_hbm, out_hbm, idx_v, rows_v, sem):
        wid = lax.axis_index("s") * info.num_cores + lax.axis_index("c")
        base = wid * b_per_w
        pltpu.sync_copy(idx_hbm.at[pl.ds(base, b_per_w)], idx_v)
        pltpu.async_copy(table_hbm.at[idx_v], rows_v, sem).wait()
        pltpu.sync_copy(rows_v, out_hbm.at[pl.ds(base, b_per_w)])

    return k(table, idx)


def _moe_body(meta_ref, x_ref, w1_ref, b1_ref, w2_ref, b2_ref, eid_ref,
              wrow_ref, out_ref):
    s = pl.program_id(0)
    e = meta_ref[1, s]
    first = meta_ref[2, s]
    valid = meta_ref[3, s]

    @pl.when(first == 1)
    def _():
        out_ref[...] = jnp.zeros_like(out_ref)

    @pl.when(valid == 1)
    def _():
        x = x_ref[...].astype(jnp.bfloat16)
        h = jnp.dot(x, w1_ref[0].astype(jnp.bfloat16),
                    preferred_element_type=jnp.float32)
        h = jax.nn.gelu(h + b1_ref[0])
        y = jnp.dot(h.astype(jnp.bfloat16), w2_ref[0].astype(jnp.bfloat16),
                    preferred_element_type=jnp.float32)
        y = y + b2_ref[0]
        coef = jnp.where(eid_ref[0, 0, :] == e, wrow_ref[0, 0, :], 0.0)
        out_ref[...] += y * coef[:, None]


def _grouped_mlp(meta, x_sorted, W1, b1, W2, b2, eid3, w3, n_steps):
    t, h = x_sorted.shape
    e, _, f = W1.shape
    grid_spec = pltpu.PrefetchScalarGridSpec(
        num_scalar_prefetch=1,
        grid=(n_steps,),
        in_specs=[
            pl.BlockSpec((TILE, h), lambda s, m: (m[0, s], 0)),
            pl.BlockSpec((1, h, f), lambda s, m: (m[1, s], 0, 0)),
            pl.BlockSpec((1, 1, f), lambda s, m: (m[1, s], 0, 0)),
            pl.BlockSpec((1, f, h), lambda s, m: (m[1, s], 0, 0)),
            pl.BlockSpec((1, 1, h), lambda s, m: (m[1, s], 0, 0)),
            pl.BlockSpec((1, 1, TILE), lambda s, m: (m[0, s], 0, 0)),
            pl.BlockSpec((1, 1, TILE), lambda s, m: (m[0, s], 0, 0)),
        ],
        out_specs=pl.BlockSpec((TILE, h), lambda s, m: (m[0, s], 0)),
    )
    return pl.pallas_call(
        _moe_body,
        grid_spec=grid_spec,
        out_shape=jax.ShapeDtypeStruct((t, h), jnp.float32),
    )(meta, x_sorted, W1, b1.reshape(e, 1, f), W2, b2.reshape(e, 1, h),
      eid3, w3)


def _routing(eid, wcomb, n_experts, n_tiles, n_steps):
    t = eid.shape[0]
    i32 = jnp.int32
    toks = jnp.arange(t, dtype=i32)
    oh = (eid[:, None] == jnp.arange(n_experts, dtype=i32)[None, :])
    oh = oh.astype(i32)
    counts = oh.sum(axis=0)
    offsets = jnp.concatenate(
        [jnp.zeros((1,), i32), jnp.cumsum(counts)[:-1].astype(i32)])
    csum = jnp.cumsum(oh, axis=0) - oh
    rank = ((offsets[None, :] + csum) * oh).sum(axis=1).astype(i32)
    payload = jnp.stack(
        [toks, eid, lax.bitcast_convert_type(wcomb, i32)], axis=1)
    sorted_payload = jnp.zeros((t, 3), i32).at[rank].set(payload)
    sort_idx = sorted_payload[:, 0]
    eid_sorted = sorted_payload[:, 1]
    w_sorted = lax.bitcast_convert_type(sorted_payload[:, 2], jnp.float32)

    er = eid_sorted.reshape(n_tiles, -1)
    first_e = er[:, 0]
    last_e = er[:, -1]
    nsteps = last_e - first_e + 1
    step_start = jnp.concatenate(
        [jnp.zeros((1,), i32), jnp.cumsum(nsteps)[:-1].astype(i32)])
    s_real = step_start[-1] + nsteps[-1]
    s_ar = jnp.arange(n_steps, dtype=i32)
    t_s = jnp.searchsorted(step_start, s_ar, side="right").astype(i32) - 1
    e_s = first_e[t_s] + (s_ar - step_start[t_s])
    valid = s_ar < s_real
    e_s = jnp.where(valid, e_s, last_e[-1]).astype(i32)
    first_flag = (valid & (s_ar == step_start[t_s])).astype(i32)
    meta = jnp.stack([t_s, e_s, first_flag, valid.astype(i32)])
    return meta, rank, sort_idx, eid_sorted, w_sorted


def kernel(hidden_states, gate_w, alpha, W1, b1, W2, b2):
    t, h = hidden_states.shape
    e = gate_w.shape[1]
    nt = t // TILE
    n_steps = nt + e - 1

    eid, wcomb = _gate(hidden_states, gate_w, alpha)
    meta, rank, sort_idx, eid_sorted, w_sorted = _routing(
        eid, wcomb, e, nt, n_steps)
    x_sorted = _sc_gather_rows(hidden_states, sort_idx)
    y_sorted = _grouped_mlp(
        meta, x_sorted, W1, b1, W2, b2,
        eid_sorted.reshape(nt, 1, TILE), w_sorted.reshape(nt, 1, TILE),
        n_steps)
    return _sc_gather_rows(y_sorted, rank)


# speedup vs baseline: 2.0121x; 1.0323x over previous
import functools

import jax
import jax.numpy as jnp
from jax import lax
from jax.experimental import pallas as pl
from jax.experimental.pallas import tpu as pltpu
from jax.experimental.pallas import tpu_sc as plsc

TILE = 128


def _gate_body(x_ref, gw_ref, alpha_ref, eid_ref, w_ref, *, n_experts):
    x = x_ref[...]
    logits = jnp.dot(x, gw_ref[...], preferred_element_type=jnp.float32)
    t, lanes = logits.shape
    col = lax.broadcasted_iota(jnp.int32, (t, lanes), 1)
    in_cols = col < n_experts
    logits = jnp.where(in_cols, logits, -1e30)
    lmax = jnp.max(logits, axis=1, keepdims=True)
    ssum = jnp.sum(jnp.where(in_cols, jnp.exp(logits - lmax), 0.0), axis=1,
                   keepdims=True)
    eidx = jnp.min(jnp.where(logits == lmax, col, n_experts), axis=1)
    alpha_sel = jnp.sum(
        jnp.where(col == eidx[:, None], alpha_ref[...], 0.0), axis=1)
    eid_ref[...] = eidx
    w_ref[...] = alpha_sel / ssum[:, 0]


def _gate(x, gate_w, alpha):
    t, h = x.shape
    e = gate_w.shape[1]
    gwp = jnp.pad(gate_w, ((0, 0), (0, 128 - e)))
    alphap = jnp.pad(alpha, (0, 128 - e)).reshape(1, 128)
    return pl.pallas_call(
        functools.partial(_gate_body, n_experts=e),
        out_shape=(jax.ShapeDtypeStruct((t,), jnp.int32),
                   jax.ShapeDtypeStruct((t,), jnp.float32)),
    )(x, gwp, alphap)


def _sc_gather_rows(table, idx):
    t, h = table.shape
    info = plsc.get_sparse_core_info()
    nw = info.num_cores * info.num_subcores
    b_per_w = t // nw
    mesh = plsc.VectorSubcoreMesh(core_axis_name="c", subcore_axis_name="s")

    @functools.partial(
        pl.kernel, mesh=mesh,
        out_type=jax.ShapeDtypeStruct((t, h), table.dtype),
        scratch_types=[
            pltpu.VMEM((b_per_w,), jnp.int32),
            pltpu.VMEM((b_per_w, h), table.dtype),
            pltpu.SemaphoreType.DMA,
        ],
    )
    def k(table_hbm, idx_hbm, out_hbm, idx_v, rows_v, sem):
        wid = lax.axis_index("s") * info.num_cores + lax.axis_index("c")
        base = wid * b_per_w
        pltpu.sync_copy(idx_hbm.at[pl.ds(base, b_per_w)], idx_v)
        pltpu.async_copy(table_hbm.at[idx_v], rows_v, sem).wait()
        pltpu.sync_copy(rows_v, out_hbm.at[pl.ds(base, b_per_w)])

    return k(table, idx)


NF = 4


def _moe_body(meta_ref, x_ref, w1_ref, b1_ref, w2_ref, b2_ref, eid_ref,
              wrow_ref, out_ref):
    e = pl.program_id(0)
    fc = pl.program_id(1)
    tile_lo = meta_ref[0, e]
    tile_hi = meta_ref[1, e]

    @pl.when((e == 0) & (fc == 0))
    def _():
        out_ref[...] = jnp.zeros_like(out_ref)

    b2_scale = jnp.where(fc == 0, 1.0, 0.0)

    def tile_step(j, _):
        rows = pl.ds(j * TILE, TILE)
        x = x_ref[rows, :].astype(jnp.bfloat16)
        h = jnp.dot(x, w1_ref[0].astype(jnp.bfloat16),
                    preferred_element_type=jnp.float32)
        h = jax.nn.gelu(h + b1_ref[0])
        y = jnp.dot(h.astype(jnp.bfloat16), w2_ref[0].astype(jnp.bfloat16),
                    preferred_element_type=jnp.float32)
        y = y + b2_scale * b2_ref[0]
        coef = jnp.where(eid_ref[rows] == e, wrow_ref[rows], 0.0)
        out_ref[rows, :] += y * coef[:, None]
        return 0

    lax.fori_loop(tile_lo, tile_hi + 1, tile_step, 0)


def _grouped_mlp(meta, x_sorted, W1, b1, W2, b2, eid_sorted, w_sorted):
    t, h = x_sorted.shape
    e, _, f = W1.shape
    fchunk = f // NF
    grid_spec = pltpu.PrefetchScalarGridSpec(
        num_scalar_prefetch=1,
        grid=(e, NF),
        in_specs=[
            pl.BlockSpec((t, h), lambda ei, fc, m: (0, 0)),
            pl.BlockSpec((1, h, fchunk), lambda ei, fc, m: (ei, 0, fc)),
            pl.BlockSpec((1, 1, fchunk), lambda ei, fc, m: (ei, 0, fc)),
            pl.BlockSpec((1, fchunk, h), lambda ei, fc, m: (ei, fc, 0)),
            pl.BlockSpec((1, 1, h), lambda ei, fc, m: (ei, 0, 0)),
            pl.BlockSpec((t,), lambda ei, fc, m: (0,)),
            pl.BlockSpec((t,), lambda ei, fc, m: (0,)),
        ],
        out_specs=pl.BlockSpec((t, h), lambda ei, fc, m: (0, 0)),
    )
    return pl.pallas_call(
        _moe_body,
        grid_spec=grid_spec,
        out_shape=jax.ShapeDtypeStruct((t, h), jnp.float32),
    )(meta, x_sorted, W1, b1.reshape(e, 1, f), W2, b2.reshape(e, 1, h),
      eid_sorted, w_sorted)


def _routing(eid, wcomb, n_experts):
    t = eid.shape[0]
    i32 = jnp.int32
    toks = jnp.arange(t, dtype=i32)
    oh = (eid[:, None] == jnp.arange(n_experts, dtype=i32)[None, :])
    oh = oh.astype(i32)
    counts = oh.sum(axis=0)
    offsets = jnp.concatenate(
        [jnp.zeros((1,), i32), jnp.cumsum(counts)[:-1].astype(i32)])
    csum = jnp.cumsum(oh, axis=0) - oh
    rank = ((offsets[None, :] + csum) * oh).sum(axis=1).astype(i32)
    payload = jnp.stack(
        [toks, eid, lax.bitcast_convert_type(wcomb, i32)], axis=1)
    sorted_payload = jnp.zeros((t, 3), i32).at[rank].set(payload)
    sort_idx = sorted_payload[:, 0]
    eid_sorted = sorted_payload[:, 1]
    w_sorted = lax.bitcast_convert_type(sorted_payload[:, 2], jnp.float32)

    tile_lo = jnp.where(counts > 0, offsets // TILE, 1).astype(i32)
    tile_hi = jnp.where(counts > 0, (offsets + counts - 1) // TILE, 0)
    meta = jnp.stack([tile_lo, tile_hi.astype(i32)])
    return meta, rank, sort_idx, eid_sorted, w_sorted


def kernel(hidden_states, gate_w, alpha, W1, b1, W2, b2):
    e = gate_w.shape[1]

    eid, wcomb = _gate(hidden_states, gate_w, alpha)
    meta, rank, sort_idx, eid_sorted, w_sorted = _routing(eid, wcomb, e)
    x_sorted = _sc_gather_rows(hidden_states, sort_idx)
    y_sorted = _grouped_mlp(meta, x_sorted, W1, b1, W2, b2, eid_sorted,
                            w_sorted)
    return _sc_gather_rows(y_sorted, rank)


# speedup vs baseline: 2.2533x; 1.1199x over previous
import functools

import jax
import jax.numpy as jnp
from jax import lax
from jax.experimental import pallas as pl
from jax.experimental.pallas import tpu as pltpu
from jax.experimental.pallas import tpu_sc as plsc

TILE = 128


def _gate_body(x_ref, gw_ref, alpha_ref, eid_ref, w_ref, *, n_experts):
    x = x_ref[...]
    logits = jnp.dot(x, gw_ref[...], preferred_element_type=jnp.float32)
    t, lanes = logits.shape
    col = lax.broadcasted_iota(jnp.int32, (t, lanes), 1)
    in_cols = col < n_experts
    logits = jnp.where(in_cols, logits, -1e30)
    lmax = jnp.max(logits, axis=1, keepdims=True)
    ssum = jnp.sum(jnp.where(in_cols, jnp.exp(logits - lmax), 0.0), axis=1,
                   keepdims=True)
    eidx = jnp.min(jnp.where(logits == lmax, col, n_experts), axis=1)
    alpha_sel = jnp.sum(
        jnp.where(col == eidx[:, None], alpha_ref[...], 0.0), axis=1)
    eid_ref[...] = eidx
    w_ref[...] = alpha_sel / ssum[:, 0]


def _gate(x, gate_w, alpha):
    t, h = x.shape
    e = gate_w.shape[1]
    gwp = jnp.pad(gate_w, ((0, 0), (0, 128 - e)))
    alphap = jnp.pad(alpha, (0, 128 - e)).reshape(1, 128)
    return pl.pallas_call(
        functools.partial(_gate_body, n_experts=e),
        out_shape=(jax.ShapeDtypeStruct((t,), jnp.int32),
                   jax.ShapeDtypeStruct((t,), jnp.float32)),
    )(x, gwp, alphap)


def _sc_gather_rows(table, idx):
    t, h = table.shape
    info = plsc.get_sparse_core_info()
    nw = info.num_cores * info.num_subcores
    b_per_w = t // nw
    mesh = plsc.VectorSubcoreMesh(core_axis_name="c", subcore_axis_name="s")

    @functools.partial(
        pl.kernel, mesh=mesh,
        out_type=jax.ShapeDtypeStruct((t, h), table.dtype),
        scratch_types=[
            pltpu.VMEM((b_per_w,), jnp.int32),
            pltpu.VMEM((b_per_w, h), table.dtype),
            pltpu.SemaphoreType.DMA,
        ],
    )
    def k(table_hbm, idx_hbm, out_hbm, idx_v, rows_v, sem):
        wid = lax.axis_index("s") * info.num_cores + lax.axis_index("c")
        base = wid * b_per_w
        pltpu.sync_copy(idx_hbm.at[pl.ds(base, b_per_w)], idx_v)
        pltpu.async_copy(table_hbm.at[idx_v], rows_v, sem).wait()
        pltpu.sync_copy(rows_v, out_hbm.at[pl.ds(base, b_per_w)])

    return k(table, idx)


NF = 4


def _moe_body(meta_ref, x_ref, w1a_ref, w1b_ref, b1_ref, w2a_ref, w2b_ref,
              b2_ref, eid_ref, wrow_ref, out_ref):
    e = pl.program_id(0)
    fc = pl.program_id(1)
    tile_lo = meta_ref[0, e]
    tile_hi = meta_ref[1, e]

    @pl.when((e == 0) & (fc == 0))
    def _():
        out_ref[...] = jnp.zeros_like(out_ref)

    b2_scale = jnp.where(fc == 0, 1.0, 0.0)
    fch = b1_ref.shape[-1] // 2

    def tile_step(j, _):
        rows = pl.ds(j * TILE, TILE)
        x = x_ref[rows, :].astype(jnp.bfloat16)
        ha = jnp.dot(x, w1a_ref[0].astype(jnp.bfloat16),
                     preferred_element_type=jnp.float32)
        hb = jnp.dot(x, w1b_ref[0].astype(jnp.bfloat16),
                     preferred_element_type=jnp.float32)
        ha = jax.nn.gelu(ha + b1_ref[0, :, :fch])
        hb = jax.nn.gelu(hb + b1_ref[0, :, fch:])
        y = jnp.dot(ha.astype(jnp.bfloat16), w2a_ref[0].astype(jnp.bfloat16),
                    preferred_element_type=jnp.float32)
        y += jnp.dot(hb.astype(jnp.bfloat16), w2b_ref[0].astype(jnp.bfloat16),
                     preferred_element_type=jnp.float32)
        y = y + b2_scale * b2_ref[0]
        coef = jnp.where(eid_ref[rows] == e, wrow_ref[rows], 0.0)
        out_ref[rows, :] += y * coef[:, None]
        return 0

    lax.fori_loop(tile_lo, tile_hi + 1, tile_step, 0)


def _grouped_mlp(meta, x_sorted, W1, b1, W2, b2, eid_sorted, w_sorted):
    t, h = x_sorted.shape
    e, _, f = W1.shape
    fchunk = f // NF
    grid_spec = pltpu.PrefetchScalarGridSpec(
        num_scalar_prefetch=1,
        grid=(e, NF // 2),
        in_specs=[
            pl.BlockSpec((t, h), lambda ei, fc, m: (0, 0)),
            pl.BlockSpec((1, h, fchunk), lambda ei, fc, m: (ei, 0, 2 * fc)),
            pl.BlockSpec((1, h, fchunk),
                         lambda ei, fc, m: (ei, 0, 2 * fc + 1)),
            pl.BlockSpec((1, 1, 2 * fchunk), lambda ei, fc, m: (ei, 0, fc)),
            pl.BlockSpec((1, fchunk, h), lambda ei, fc, m: (ei, 2 * fc, 0)),
            pl.BlockSpec((1, fchunk, h),
                         lambda ei, fc, m: (ei, 2 * fc + 1, 0)),
            pl.BlockSpec((1, 1, h), lambda ei, fc, m: (ei, 0, 0)),
            pl.BlockSpec((t,), lambda ei, fc, m: (0,)),
            pl.BlockSpec((t,), lambda ei, fc, m: (0,)),
        ],
        out_specs=pl.BlockSpec((t, h), lambda ei, fc, m: (0, 0)),
    )
    return pl.pallas_call(
        _moe_body,
        grid_spec=grid_spec,
        out_shape=jax.ShapeDtypeStruct((t, h), jnp.float32),
    )(meta, x_sorted, W1, W1, b1.reshape(e, 1, f), W2, W2,
      b2.reshape(e, 1, h), eid_sorted, w_sorted)


def _routing(eid, wcomb, n_experts):
    t = eid.shape[0]
    i32 = jnp.int32
    toks = jnp.arange(t, dtype=i32)
    oh = (eid[:, None] == jnp.arange(n_experts, dtype=i32)[None, :])
    oh = oh.astype(i32)
    counts = oh.sum(axis=0)
    offsets = jnp.concatenate(
        [jnp.zeros((1,), i32), jnp.cumsum(counts)[:-1].astype(i32)])
    csum = jnp.cumsum(oh, axis=0) - oh
    rank = ((offsets[None, :] + csum) * oh).sum(axis=1).astype(i32)
    payload = jnp.stack(
        [toks, eid, lax.bitcast_convert_type(wcomb, i32)], axis=1)
    sorted_payload = jnp.zeros((t, 3), i32).at[rank].set(payload)
    sort_idx = sorted_payload[:, 0]
    eid_sorted = sorted_payload[:, 1]
    w_sorted = lax.bitcast_convert_type(sorted_payload[:, 2], jnp.float32)

    tile_lo = jnp.where(counts > 0, offsets // TILE, 1).astype(i32)
    tile_hi = jnp.where(counts > 0, (offsets + counts - 1) // TILE, 0)
    meta = jnp.stack([tile_lo, tile_hi.astype(i32)])
    return meta, rank, sort_idx, eid_sorted, w_sorted


def kernel(hidden_states, gate_w, alpha, W1, b1, W2, b2):
    e = gate_w.shape[1]

    eid, wcomb = _gate(hidden_states, gate_w, alpha)
    meta, rank, sort_idx, eid_sorted, w_sorted = _routing(eid, wcomb, e)
    x_sorted = _sc_gather_rows(hidden_states, sort_idx)
    y_sorted = _grouped_mlp(meta, x_sorted, W1, b1, W2, b2, eid_sorted,
                            w_sorted)
    return _sc_gather_rows(y_sorted, rank)
